# baseline (speedup 1.0000x reference)
import functools

import jax
import jax.numpy as jnp
from jax import lax
from jax.experimental import pallas as pl
from jax.experimental.pallas import tpu as pltpu
from jax.experimental.pallas import tpu_sc as plsc

_INFO = plsc.get_sparse_core_info()
_NC, _NS = _INFO.num_cores, _INFO.num_subcores
_NW = _NC * _NS

_B = 4 * 8192
_D = 64
_B_PER_W = _B // _NW
_CHUNK = 256
_NCHUNK = _B_PER_W // _CHUNK
_NBUF = 2


def _gather_body(pid_hbm, cos_hbm, sin_hbm, cos_out, sin_out,
                 idx_v, cos_b, sin_b, g0, g1, o0, o1):
    wid = lax.axis_index("s") * _NC + lax.axis_index("c")
    base = wid * _B_PER_W
    pltpu.sync_copy(pid_hbm.at[pl.ds(base, _B_PER_W)], idx_v)
    gsem, osem = [g0, g1], [o0, o1]
    gcp = [None] * _NCHUNK
    ocp = [None] * _NCHUNK

    def start_gather(c):
        b = c % _NBUF
        ii = idx_v.at[pl.ds(c * _CHUNK, _CHUNK)]
        gcp[c] = (pltpu.async_copy(cos_hbm.at[ii], cos_b.at[b], gsem[b]),
                  pltpu.async_copy(sin_hbm.at[ii], sin_b.at[b], gsem[b]))

    def start_out(c):
        b = c % _NBUF
        lo = base + c * _CHUNK
        for cp in gcp[c]:
            cp.wait()
        ocp[c] = (pltpu.async_copy(cos_b.at[b], cos_out.at[pl.ds(lo, _CHUNK)], osem[b]),
                  pltpu.async_copy(sin_b.at[b], sin_out.at[pl.ds(lo, _CHUNK)], osem[b]))

    start_gather(0)
    for c in range(1, _NCHUNK):
        start_out(c - 1)
        if c >= _NBUF:
            for cp in ocp[c - _NBUF]:
                cp.wait()
        start_gather(c)
    start_out(_NCHUNK - 1)
    for c in range(max(0, _NCHUNK - _NBUF), _NCHUNK):
        for cp in ocp[c]:
            cp.wait()


@functools.partial(jax.jit, static_argnames=())
def _rope_gather(position_ids_flat, cos_cached, sin_cached):
    mesh = plsc.VectorSubcoreMesh(core_axis_name="c", subcore_axis_name="s")
    k = pl.kernel(
        _gather_body,
        out_type=[
            jax.ShapeDtypeStruct((_B, _D), jnp.float32),
            jax.ShapeDtypeStruct((_B, _D), jnp.float32),
        ],
        mesh=mesh,
        scratch_types=[
            pltpu.VMEM((_B_PER_W,), jnp.int32),
            pltpu.VMEM((_NBUF, _CHUNK, _D), jnp.float32),
            pltpu.VMEM((_NBUF, _CHUNK, _D), jnp.float32),
            pltpu.SemaphoreType.DMA,
            pltpu.SemaphoreType.DMA,
            pltpu.SemaphoreType.DMA,
            pltpu.SemaphoreType.DMA,
        ],
        compiler_params=pltpu.CompilerParams(use_tc_tiling_on_sc=False),
    )
    return k(position_ids_flat, cos_cached, sin_cached)


def kernel(x, position_ids, cos_cached, sin_cached):
    b, s = position_ids.shape
    pid = position_ids.reshape(b * s)
    cos, sin = _rope_gather(pid, cos_cached, sin_cached)
    cos = cos.reshape(b, s, _D).astype(x.dtype)
    sin = sin.reshape(b, s, _D).astype(x.dtype)
    return (cos, sin)
